# Initial kernel scaffold; baseline (speedup 1.0000x reference)
import jax
import jax.numpy as jnp
from jax.experimental import pallas as pl


def kernel(x, edge_index, W1, b1, W2, b2, W_ro, b_ro):
    raise NotImplementedError("write your pallas kernel here")


# speedup vs baseline: 5.0504x; 5.0504x over previous
import dataclasses
import functools

import jax
import jax.numpy as jnp
from jax import lax
from jax.experimental import pallas as pl
from jax.experimental.pallas import tpu as pltpu
from jax.experimental.pallas import tpu_sc as plsc

N = 10000
NP = 10240
E = 320000
D = 128

NC = 2
NS = 16
LANES = 16
E_PER_CORE = E // NC
E_PER_TILE = E_PER_CORE // NS
CK = 80
NCHUNK = E_PER_TILE // CK
ROWS_PER_TILE = NP // NS
ZROWS = 128

_MESH = plsc.VectorSubcoreMesh(core_axis_name="c", subcore_axis_name="s")

_SC_CP = pltpu.CompilerParams()
if "needs_layout_passes" in pltpu.CompilerParams.__dataclass_fields__:
    _SC_CP = dataclasses.replace(_SC_CP, needs_layout_passes=False)


NW = NC * NS


def _sc_degrees(src, dst):
    @functools.partial(
        pl.kernel,
        out_type=(
            jax.ShapeDtypeStruct((NW, NP), jnp.float32),
            jax.ShapeDtypeStruct((NW, NP), jnp.float32),
        ),
        mesh=_MESH,
        scratch_types=[
            pltpu.VMEM((CK,), jnp.int32),
            pltpu.VMEM((CK,), jnp.int32),
            pltpu.VMEM((NP,), jnp.float32),
            pltpu.VMEM((NP,), jnp.float32),
        ],
        compiler_params=_SC_CP,
    )
    def k(src_hbm, dst_hbm, odeg_hbm, ideg_hbm, idx_s, idx_d, hist_o, hist_i):
        core = lax.axis_index("c")
        sub = lax.axis_index("s")
        wid = core * NS + sub

        @pl.loop(0, NP // LANES)
        def _(r):
            z = jnp.zeros((LANES,), jnp.float32)
            hist_o[pl.ds(r * LANES, LANES)] = z
            hist_i[pl.ds(r * LANES, LANES)] = z

        base = core * E_PER_CORE + sub * E_PER_TILE
        ones16 = jnp.ones((LANES,), jnp.float32)

        @pl.loop(0, NCHUNK)
        def _(ci):
            off = base + ci * CK
            pltpu.sync_copy(src_hbm.at[pl.ds(off, CK)], idx_s)
            pltpu.sync_copy(dst_hbm.at[pl.ds(off, CK)], idx_d)
            for j in range(CK // LANES):
                sl = pl.ds(j * LANES, LANES)
                plsc.addupdate_scatter(hist_o, [idx_s[sl]], ones16)
                plsc.addupdate_scatter(hist_i, [idx_d[sl]], ones16)

        pltpu.sync_copy(hist_o, odeg_hbm.at[wid])
        pltpu.sync_copy(hist_i, ideg_hbm.at[wid])

    return k(src, dst)


def _sc_spmm(xp, src, dst):
    @functools.partial(
        pl.kernel,
        out_type=jax.ShapeDtypeStruct((NC, NP, D), jnp.float32),
        mesh=_MESH,
        scratch_types=[
            pltpu.VMEM((CK,), jnp.int32),
            pltpu.VMEM((CK,), jnp.int32),
            pltpu.VMEM((CK, D), jnp.float32),
            pltpu.VMEM((ZROWS, D), jnp.float32),
            pltpu.VMEM_SHARED((NP, D), jnp.float32),
            pltpu.SemaphoreType.DMA,
        ],
    )
    def k(xp_hbm, src_hbm, dst_hbm, out_hbm, idx_s, idx_d, rows_v, slab_v,
          acc, sem):
        core = lax.axis_index("c")
        sub = lax.axis_index("s")
        row0 = sub * ROWS_PER_TILE

        @pl.loop(0, ZROWS)
        def _(r):
            for cc in range(D // LANES):
                slab_v[r, pl.ds(cc * LANES, LANES)] = jnp.zeros(
                    (LANES,), jnp.float32)

        for kk in range(ROWS_PER_TILE // ZROWS):
            pltpu.sync_copy(slab_v, acc.at[pl.ds(row0 + kk * ZROWS, ZROWS)])
        plsc.subcore_barrier()

        base = core * E_PER_CORE + sub * E_PER_TILE

        @pl.loop(0, NCHUNK)
        def _(ci):
            off = base + ci * CK
            pltpu.sync_copy(src_hbm.at[pl.ds(off, CK)], idx_s)
            pltpu.sync_copy(dst_hbm.at[pl.ds(off, CK)], idx_d)
            pltpu.async_copy(xp_hbm.at[idx_s], rows_v, sem).wait()
            pltpu.sync_copy(rows_v, acc.at[idx_d], add=True)

        plsc.subcore_barrier()

        for kk in range(ROWS_PER_TILE // ZROWS):
            r0 = row0 + kk * ZROWS
            pltpu.sync_copy(acc.at[pl.ds(r0, ZROWS)], slab_v)
            pltpu.sync_copy(slab_v, out_hbm.at[core, pl.ds(r0, ZROWS)])

    return k(xp, src, dst)


BR = 1024


def _deg_vec(d_ref):
    return jnp.sum(d_ref[...], axis=0)


def _tc_scale_body(x_ref, od_ref, o_ref):
    ns = lax.rsqrt(jnp.maximum(_deg_vec(od_ref), 1.0))
    o_ref[...] = x_ref[...] * ns[:, None]


def _tc_layer1_body(agg_ref, od_ref, id_ref, w_ref, b_ref, o_ref):
    agg = agg_ref[0] + agg_ref[1]
    nd = lax.rsqrt(jnp.maximum(_deg_vec(id_ref), 1.0))
    ns = lax.rsqrt(jnp.maximum(_deg_vec(od_ref), 1.0))
    h = jnp.dot(agg * nd[:, None], w_ref[...],
                preferred_element_type=jnp.float32) + b_ref[...]
    o_ref[...] = jnp.maximum(h, 0.0) * ns[:, None]


def _tc_readout_body(agg_ref, id_ref, w2_ref, b2_ref, wro_ref, bro_ref,
                     o_ref, acc_ref):
    i = pl.program_id(0)
    agg = agg_ref[0] + agg_ref[1]
    nd = lax.rsqrt(jnp.maximum(_deg_vec(id_ref), 1.0))
    part = jnp.sum(agg * nd[:, None], axis=0, keepdims=True)

    @pl.when(i == 0)
    def _():
        acc_ref[...] = part

    @pl.when(i > 0)
    def _():
        acc_ref[...] = acc_ref[...] + part

    @pl.when(i == pl.num_programs(0) - 1)
    def _():
        g = jnp.dot(acc_ref[...] * (1.0 / N), w2_ref[...],
                    preferred_element_type=jnp.float32) + b2_ref[...]
        z = jnp.sum(g * wro_ref[...], axis=1, keepdims=True) + bro_ref[...]
        o_ref[...] = jax.nn.sigmoid(z)


def _deg_spec():
    return pl.BlockSpec((NW, BR), lambda i: (0, i))


def _full(shape):
    return pl.BlockSpec(shape, lambda i: tuple(0 for _ in shape))


def _tc_scale(x, odeg):
    return pl.pallas_call(
        _tc_scale_body,
        grid=(NP // BR,),
        in_specs=[pl.BlockSpec((BR, D), lambda i: (i, 0)), _deg_spec()],
        out_specs=pl.BlockSpec((BR, D), lambda i: (i, 0)),
        out_shape=jax.ShapeDtypeStruct((NP, D), jnp.float32),
    )(x, odeg)


def _tc_layer1(agg, odeg, ideg, W1, b1):
    return pl.pallas_call(
        _tc_layer1_body,
        grid=(NP // BR,),
        in_specs=[
            pl.BlockSpec((NC, BR, D), lambda i: (0, i, 0)),
            _deg_spec(), _deg_spec(),
            _full((D, D)), _full((1, D)),
        ],
        out_specs=pl.BlockSpec((BR, D), lambda i: (i, 0)),
        out_shape=jax.ShapeDtypeStruct((NP, D), jnp.float32),
    )(agg, odeg, ideg, W1, b1)


def _tc_readout(agg, ideg, W2, b2, W_ro, b_ro):
    return pl.pallas_call(
        _tc_readout_body,
        grid=(NP // BR,),
        in_specs=[
            pl.BlockSpec((NC, BR, D), lambda i: (0, i, 0)),
            _deg_spec(),
            _full((D, D)), _full((1, D)), _full((1, D)), _full((1, 1)),
        ],
        out_specs=_full((1, 1)),
        out_shape=jax.ShapeDtypeStruct((1, 1), jnp.float32),
        scratch_shapes=[pltpu.VMEM((1, D), jnp.float32)],
    )(agg, ideg, W2, b2, W_ro, b_ro)


def kernel(x, edge_index, W1, b1, W2, b2, W_ro, b_ro):
    src = edge_index[0]
    dst = edge_index[1]
    odeg, ideg = _sc_degrees(src, dst)
    x_pad = jnp.pad(x, ((0, NP - N), (0, 0)))
    xp = _tc_scale(x_pad, odeg)
    agg1 = _sc_spmm(xp, src, dst)
    h1p = _tc_layer1(agg1, odeg, ideg, W1, b1.reshape(1, D))
    agg2 = _sc_spmm(h1p, src, dst)
    out = _tc_readout(agg2, ideg, W2, b2.reshape(1, D),
                      W_ro.reshape(1, D), b_ro.reshape(1, 1))
    return jnp.squeeze(out)


# speedup vs baseline: 10.5994x; 2.0987x over previous
import dataclasses
import functools

import jax
import jax.numpy as jnp
from jax import lax
from jax.experimental import pallas as pl
from jax.experimental.pallas import tpu as pltpu
from jax.experimental.pallas import tpu_sc as plsc

N = 10000
NP = 10240
E = 320000
D = 128

NC = 2
NS = 16
LANES = 16
E_PER_CORE = E // NC
E_PER_TILE = E_PER_CORE // NS
CK = 80
NCHUNK = E_PER_TILE // CK
ROWS_PER_TILE = NP // NS
ZROWS = 128

_MESH = plsc.VectorSubcoreMesh(core_axis_name="c", subcore_axis_name="s")

_SC_CP = pltpu.CompilerParams()
if "needs_layout_passes" in pltpu.CompilerParams.__dataclass_fields__:
    _SC_CP = dataclasses.replace(_SC_CP, needs_layout_passes=False)


NW = NC * NS


def _sc_degrees(src, dst):
    @functools.partial(
        pl.kernel,
        out_type=(
            jax.ShapeDtypeStruct((NW, NP), jnp.float32),
            jax.ShapeDtypeStruct((NW, NP), jnp.float32),
        ),
        mesh=_MESH,
        scratch_types=[
            pltpu.VMEM((E_PER_TILE,), jnp.int32),
            pltpu.VMEM((E_PER_TILE,), jnp.int32),
            pltpu.VMEM((NP,), jnp.float32),
            pltpu.VMEM((NP,), jnp.float32),
            pltpu.SemaphoreType.DMA,
            pltpu.SemaphoreType.DMA,
        ],
        compiler_params=_SC_CP,
    )
    def k(src_hbm, dst_hbm, odeg_hbm, ideg_hbm, sidx, didx, hist_o, hist_i,
          sem_a, sem_b):
        core = lax.axis_index("c")
        sub = lax.axis_index("s")
        wid = core * NS + sub
        base = wid * E_PER_TILE

        pltpu.async_copy(src_hbm.at[pl.ds(base, E_PER_TILE)], sidx, sem_a)
        pltpu.async_copy(dst_hbm.at[pl.ds(base, E_PER_TILE)], didx, sem_b)

        @pl.loop(0, NP // LANES)
        def _(r):
            z = jnp.zeros((LANES,), jnp.float32)
            hist_o[pl.ds(r * LANES, LANES)] = z
            hist_i[pl.ds(r * LANES, LANES)] = z

        pltpu.make_async_copy(src_hbm.at[pl.ds(base, E_PER_TILE)], sidx,
                              sem_a).wait()
        pltpu.make_async_copy(dst_hbm.at[pl.ds(base, E_PER_TILE)], didx,
                              sem_b).wait()
        ones16 = jnp.ones((LANES,), jnp.float32)

        @pl.loop(0, E_PER_TILE // LANES)
        def _(g):
            sl = pl.ds(g * LANES, LANES)
            plsc.addupdate_scatter(hist_o, [sidx[sl]], ones16)
            plsc.addupdate_scatter(hist_i, [didx[sl]], ones16)

        pltpu.sync_copy(hist_o, odeg_hbm.at[wid])
        pltpu.sync_copy(hist_i, ideg_hbm.at[wid])

    return k(src, dst)


RCHUNK = ROWS_PER_TILE // CK


def _sc_spmm(xp, src, dst3d):
    @functools.partial(
        pl.kernel,
        out_type=jax.ShapeDtypeStruct((NC, NP, D), jnp.float32),
        mesh=_MESH,
        scratch_types=[
            pltpu.VMEM((E_PER_TILE,), jnp.int32),
            pltpu.VMEM((NCHUNK, CK), jnp.int32),
            pltpu.VMEM((CK, D), jnp.float32),
            pltpu.VMEM((CK, D), jnp.float32),
            pltpu.VMEM_SHARED((NP, D), jnp.float32),
            pltpu.SemaphoreType.DMA,
            pltpu.SemaphoreType.DMA,
        ],
    )
    def k(xp_hbm, src_hbm, dst_hbm, out_hbm, sidx, didx, rows_a, rows_b,
          acc, sem_a, sem_b):
        core = lax.axis_index("c")
        sub = lax.axis_index("s")
        wid = core * NS + sub
        row0 = sub * ROWS_PER_TILE
        base = wid * E_PER_TILE

        pltpu.async_copy(src_hbm.at[pl.ds(base, E_PER_TILE)], sidx, sem_a)
        pltpu.async_copy(dst_hbm.at[wid], didx, sem_b)

        @pl.loop(0, CK)
        def _(r):
            for cc in range(D // LANES):
                rows_a[r, pl.ds(cc * LANES, LANES)] = jnp.zeros(
                    (LANES,), jnp.float32)

        for kk in range(RCHUNK):
            pltpu.sync_copy(rows_a, acc.at[pl.ds(row0 + kk * CK, CK)])
        pltpu.make_async_copy(src_hbm.at[pl.ds(base, E_PER_TILE)], sidx,
                              sem_a).wait()
        pltpu.make_async_copy(dst_hbm.at[wid], didx, sem_b).wait()
        plsc.subcore_barrier()

        def gdesc(ci, buf, sem):
            return pltpu.make_async_copy(
                xp_hbm.at[sidx.at[pl.ds(ci * CK, CK)]], buf, sem)

        def scat(ci, buf):
            pltpu.sync_copy(buf, acc.at[didx.at[ci]], add=True)

        gdesc(0, rows_a, sem_a).start()

        @pl.loop(0, (NCHUNK - 1) // 2)
        def _(i):
            c0 = 2 * i
            gdesc(c0, rows_a, sem_a).wait()
            gdesc(c0 + 1, rows_b, sem_b).start()
            scat(c0, rows_a)
            gdesc(c0 + 1, rows_b, sem_b).wait()
            gdesc(c0 + 2, rows_a, sem_a).start()
            scat(c0 + 1, rows_b)

        gdesc(NCHUNK - 1, rows_a, sem_a).wait()
        scat(NCHUNK - 1, rows_a)

        plsc.subcore_barrier()

        for kk in range(RCHUNK):
            r0 = row0 + kk * CK
            pltpu.sync_copy(acc.at[pl.ds(r0, CK)], rows_a)
            pltpu.sync_copy(rows_a, out_hbm.at[core, pl.ds(r0, CK)])

    return k(xp, src, dst3d)


BR = 1024


def _deg_vec(d_ref):
    return jnp.sum(d_ref[...], axis=0)


def _tc_scale_body(x_ref, od_ref, o_ref):
    ns = lax.rsqrt(jnp.maximum(_deg_vec(od_ref), 1.0))
    o_ref[...] = x_ref[...] * ns[:, None]


def _tc_layer1_body(agg_ref, od_ref, id_ref, w_ref, b_ref, o_ref):
    agg = agg_ref[0] + agg_ref[1]
    nd = lax.rsqrt(jnp.maximum(_deg_vec(id_ref), 1.0))
    ns = lax.rsqrt(jnp.maximum(_deg_vec(od_ref), 1.0))
    h = jnp.dot(agg * nd[:, None], w_ref[...],
                preferred_element_type=jnp.float32) + b_ref[...]
    o_ref[...] = jnp.maximum(h, 0.0) * ns[:, None]


def _tc_readout_body(agg_ref, id_ref, w2_ref, b2_ref, wro_ref, bro_ref,
                     o_ref, acc_ref):
    i = pl.program_id(0)
    agg = agg_ref[0] + agg_ref[1]
    nd = lax.rsqrt(jnp.maximum(_deg_vec(id_ref), 1.0))
    part = jnp.sum(agg * nd[:, None], axis=0, keepdims=True)

    @pl.when(i == 0)
    def _():
        acc_ref[...] = part

    @pl.when(i > 0)
    def _():
        acc_ref[...] = acc_ref[...] + part

    @pl.when(i == pl.num_programs(0) - 1)
    def _():
        g = jnp.dot(acc_ref[...] * (1.0 / N), w2_ref[...],
                    preferred_element_type=jnp.float32) + b2_ref[...]
        z = jnp.sum(g * wro_ref[...], axis=1, keepdims=True) + bro_ref[...]
        o_ref[...] = jax.nn.sigmoid(z)


def _deg_spec():
    return pl.BlockSpec((NW, BR), lambda i: (0, i))


def _full(shape):
    return pl.BlockSpec(shape, lambda i: tuple(0 for _ in shape))


def _tc_scale(x, odeg):
    return pl.pallas_call(
        _tc_scale_body,
        grid=(NP // BR,),
        in_specs=[pl.BlockSpec((BR, D), lambda i: (i, 0)), _deg_spec()],
        out_specs=pl.BlockSpec((BR, D), lambda i: (i, 0)),
        out_shape=jax.ShapeDtypeStruct((NP, D), jnp.float32),
    )(x, odeg)


def _tc_layer1(agg, odeg, ideg, W1, b1):
    return pl.pallas_call(
        _tc_layer1_body,
        grid=(NP // BR,),
        in_specs=[
            pl.BlockSpec((NC, BR, D), lambda i: (0, i, 0)),
            _deg_spec(), _deg_spec(),
            _full((D, D)), _full((1, D)),
        ],
        out_specs=pl.BlockSpec((BR, D), lambda i: (i, 0)),
        out_shape=jax.ShapeDtypeStruct((NP, D), jnp.float32),
    )(agg, odeg, ideg, W1, b1)


def _tc_readout(agg, ideg, W2, b2, W_ro, b_ro):
    return pl.pallas_call(
        _tc_readout_body,
        grid=(NP // BR,),
        in_specs=[
            pl.BlockSpec((NC, BR, D), lambda i: (0, i, 0)),
            _deg_spec(),
            _full((D, D)), _full((1, D)), _full((1, D)), _full((1, 1)),
        ],
        out_specs=_full((1, 1)),
        out_shape=jax.ShapeDtypeStruct((1, 1), jnp.float32),
        scratch_shapes=[pltpu.VMEM((1, D), jnp.float32)],
    )(agg, ideg, W2, b2, W_ro, b_ro)


def kernel(x, edge_index, W1, b1, W2, b2, W_ro, b_ro):
    src = edge_index[0]
    dst3d = edge_index[1].reshape(NW, NCHUNK, CK)
    odeg, ideg = _sc_degrees(src, edge_index[1])
    x_pad = jnp.pad(x, ((0, NP - N), (0, 0)))
    xp = _tc_scale(x_pad, odeg)
    agg1 = _sc_spmm(xp, src, dst3d)
    h1p = _tc_layer1(agg1, odeg, ideg, W1, b1.reshape(1, D))
    agg2 = _sc_spmm(h1p, src, dst3d)
    out = _tc_readout(agg2, ideg, W2, b2.reshape(1, D),
                      W_ro.reshape(1, D), b_ro.reshape(1, 1))
    return jnp.squeeze(out)


# speedup vs baseline: 10.6769x; 1.0073x over previous
import dataclasses
import functools

import jax
import jax.numpy as jnp
from jax import lax
from jax.experimental import pallas as pl
from jax.experimental.pallas import tpu as pltpu
from jax.experimental.pallas import tpu_sc as plsc

N = 10000
NP = 10240
E = 320000
D = 128

NC = 2
NS = 16
LANES = 16
E_PER_CORE = E // NC
E_PER_TILE = E_PER_CORE // NS
CK = 80
NCHUNK = E_PER_TILE // CK
ROWS_PER_TILE = NP // NS
ZROWS = 128

_MESH = plsc.VectorSubcoreMesh(core_axis_name="c", subcore_axis_name="s")

_SC_CP = pltpu.CompilerParams()
if "needs_layout_passes" in pltpu.CompilerParams.__dataclass_fields__:
    _SC_CP = dataclasses.replace(_SC_CP, needs_layout_passes=False)


NW = NC * NS


def _sc_degrees(src, dst):
    @functools.partial(
        pl.kernel,
        out_type=(
            jax.ShapeDtypeStruct((NW, NP), jnp.float32),
            jax.ShapeDtypeStruct((NW, NP), jnp.float32),
        ),
        mesh=_MESH,
        scratch_types=[
            pltpu.VMEM((E_PER_TILE,), jnp.int32),
            pltpu.VMEM((E_PER_TILE,), jnp.int32),
            pltpu.VMEM((NP,), jnp.float32),
            pltpu.VMEM((NP,), jnp.float32),
            pltpu.SemaphoreType.DMA,
            pltpu.SemaphoreType.DMA,
        ],
        compiler_params=_SC_CP,
    )
    def k(src_hbm, dst_hbm, odeg_hbm, ideg_hbm, sidx, didx, hist_o, hist_i,
          sem_a, sem_b):
        core = lax.axis_index("c")
        sub = lax.axis_index("s")
        wid = core * NS + sub
        base = wid * E_PER_TILE

        pltpu.async_copy(src_hbm.at[pl.ds(base, E_PER_TILE)], sidx, sem_a)
        pltpu.async_copy(dst_hbm.at[pl.ds(base, E_PER_TILE)], didx, sem_b)

        @pl.loop(0, NP // LANES)
        def _(r):
            z = jnp.zeros((LANES,), jnp.float32)
            hist_o[pl.ds(r * LANES, LANES)] = z
            hist_i[pl.ds(r * LANES, LANES)] = z

        pltpu.make_async_copy(src_hbm.at[pl.ds(base, E_PER_TILE)], sidx,
                              sem_a).wait()
        pltpu.make_async_copy(dst_hbm.at[pl.ds(base, E_PER_TILE)], didx,
                              sem_b).wait()
        ones16 = jnp.ones((LANES,), jnp.float32)

        @pl.loop(0, E_PER_TILE // LANES)
        def _(g):
            sl = pl.ds(g * LANES, LANES)
            plsc.addupdate_scatter(hist_o, [sidx[sl]], ones16)
            plsc.addupdate_scatter(hist_i, [didx[sl]], ones16)

        pltpu.sync_copy(hist_o, odeg_hbm.at[wid])
        pltpu.sync_copy(hist_i, ideg_hbm.at[wid])

    return k(src, dst)


RCHUNK = ROWS_PER_TILE // CK


def _sc_spmm(xp, src, dst3d):
    @functools.partial(
        pl.kernel,
        out_type=jax.ShapeDtypeStruct((NC, NP, D), jnp.float32),
        mesh=_MESH,
        scratch_types=[
            pltpu.VMEM((E_PER_TILE,), jnp.int32),
            pltpu.VMEM((NCHUNK, CK), jnp.int32),
            pltpu.VMEM((CK, D), jnp.float32),
            pltpu.VMEM((CK, D), jnp.float32),
            pltpu.VMEM_SHARED((NP, D), jnp.float32),
            pltpu.SemaphoreType.DMA,
            pltpu.SemaphoreType.DMA,
            pltpu.SemaphoreType.DMA,
            pltpu.SemaphoreType.DMA,
        ],
    )
    def k(xp_hbm, src_hbm, dst_hbm, out_hbm, sidx, didx, rows_a, rows_b,
          acc, sem_a, sem_b, sem_sa, sem_sb):
        core = lax.axis_index("c")
        sub = lax.axis_index("s")
        wid = core * NS + sub
        row0 = sub * ROWS_PER_TILE
        base = wid * E_PER_TILE

        pltpu.async_copy(src_hbm.at[pl.ds(base, E_PER_TILE)], sidx, sem_a)
        pltpu.async_copy(dst_hbm.at[wid], didx, sem_b)

        @pl.loop(0, CK)
        def _(r):
            for cc in range(D // LANES):
                rows_a[r, pl.ds(cc * LANES, LANES)] = jnp.zeros(
                    (LANES,), jnp.float32)

        for kk in range(RCHUNK):
            pltpu.sync_copy(rows_a, acc.at[pl.ds(row0 + kk * CK, CK)])
        pltpu.make_async_copy(src_hbm.at[pl.ds(base, E_PER_TILE)], sidx,
                              sem_a).wait()
        pltpu.make_async_copy(dst_hbm.at[wid], didx, sem_b).wait()
        plsc.subcore_barrier()

        def gdesc(ci, buf, sem):
            return pltpu.make_async_copy(
                xp_hbm.at[sidx.at[pl.ds(ci * CK, CK)]], buf, sem)

        class sdesc:
            def __init__(self, ci, buf, sem):
                self.args = (buf, acc.at[didx.at[ci]], sem)

            def start(self):
                pltpu.async_copy(*self.args, add=True)

            def wait(self):
                pltpu.make_async_copy(*self.args).wait()

        gdesc(0, rows_a, sem_a).start()
        gdesc(1, rows_b, sem_b).start()

        @pl.loop(0, (NCHUNK - 3) // 2)
        def _(i):
            c0 = 2 * i
            gdesc(c0, rows_a, sem_a).wait()
            sdesc(c0, rows_a, sem_sa).start()
            gdesc(c0 + 1, rows_b, sem_b).wait()
            sdesc(c0 + 1, rows_b, sem_sb).start()
            sdesc(c0, rows_a, sem_sa).wait()
            gdesc(c0 + 2, rows_a, sem_a).start()
            sdesc(c0 + 1, rows_b, sem_sb).wait()
            gdesc(c0 + 3, rows_b, sem_b).start()

        gdesc(NCHUNK - 3, rows_a, sem_a).wait()
        sdesc(NCHUNK - 3, rows_a, sem_sa).start()
        gdesc(NCHUNK - 2, rows_b, sem_b).wait()
        sdesc(NCHUNK - 2, rows_b, sem_sb).start()
        sdesc(NCHUNK - 3, rows_a, sem_sa).wait()
        gdesc(NCHUNK - 1, rows_a, sem_a).start()
        gdesc(NCHUNK - 1, rows_a, sem_a).wait()
        sdesc(NCHUNK - 1, rows_a, sem_sa).start()
        sdesc(NCHUNK - 1, rows_a, sem_sa).wait()
        sdesc(NCHUNK - 2, rows_b, sem_sb).wait()

        plsc.subcore_barrier()

        for kk in range(RCHUNK):
            r0 = row0 + kk * CK
            pltpu.sync_copy(acc.at[pl.ds(r0, CK)], rows_a)
            pltpu.sync_copy(rows_a, out_hbm.at[core, pl.ds(r0, CK)])

    return k(xp, src, dst3d)


BR = 1024


def _deg_vec(d_ref):
    return jnp.sum(d_ref[...], axis=0)


def _tc_scale_body(x_ref, od_ref, o_ref):
    ns = lax.rsqrt(jnp.maximum(_deg_vec(od_ref), 1.0))
    o_ref[...] = x_ref[...] * ns[:, None]


def _tc_layer1_body(agg_ref, od_ref, id_ref, w_ref, b_ref, o_ref):
    agg = agg_ref[0] + agg_ref[1]
    nd = lax.rsqrt(jnp.maximum(_deg_vec(id_ref), 1.0))
    ns = lax.rsqrt(jnp.maximum(_deg_vec(od_ref), 1.0))
    h = jnp.dot(agg * nd[:, None], w_ref[...],
                preferred_element_type=jnp.float32) + b_ref[...]
    o_ref[...] = jnp.maximum(h, 0.0) * ns[:, None]


def _tc_readout_body(agg_ref, id_ref, w2_ref, b2_ref, wro_ref, bro_ref,
                     o_ref, acc_ref):
    i = pl.program_id(0)
    agg = agg_ref[0] + agg_ref[1]
    nd = lax.rsqrt(jnp.maximum(_deg_vec(id_ref), 1.0))
    part = jnp.sum(agg * nd[:, None], axis=0, keepdims=True)

    @pl.when(i == 0)
    def _():
        acc_ref[...] = part

    @pl.when(i > 0)
    def _():
        acc_ref[...] = acc_ref[...] + part

    @pl.when(i == pl.num_programs(0) - 1)
    def _():
        g = jnp.dot(acc_ref[...] * (1.0 / N), w2_ref[...],
                    preferred_element_type=jnp.float32) + b2_ref[...]
        z = jnp.sum(g * wro_ref[...], axis=1, keepdims=True) + bro_ref[...]
        o_ref[...] = jax.nn.sigmoid(z)


def _deg_spec():
    return pl.BlockSpec((NW, BR), lambda i: (0, i))


def _full(shape):
    return pl.BlockSpec(shape, lambda i: tuple(0 for _ in shape))


def _tc_scale(x, odeg):
    return pl.pallas_call(
        _tc_scale_body,
        grid=(NP // BR,),
        in_specs=[pl.BlockSpec((BR, D), lambda i: (i, 0)), _deg_spec()],
        out_specs=pl.BlockSpec((BR, D), lambda i: (i, 0)),
        out_shape=jax.ShapeDtypeStruct((NP, D), jnp.float32),
    )(x, odeg)


def _tc_layer1(agg, odeg, ideg, W1, b1):
    return pl.pallas_call(
        _tc_layer1_body,
        grid=(NP // BR,),
        in_specs=[
            pl.BlockSpec((NC, BR, D), lambda i: (0, i, 0)),
            _deg_spec(), _deg_spec(),
            _full((D, D)), _full((1, D)),
        ],
        out_specs=pl.BlockSpec((BR, D), lambda i: (i, 0)),
        out_shape=jax.ShapeDtypeStruct((NP, D), jnp.float32),
    )(agg, odeg, ideg, W1, b1)


def _tc_readout(agg, ideg, W2, b2, W_ro, b_ro):
    return pl.pallas_call(
        _tc_readout_body,
        grid=(NP // BR,),
        in_specs=[
            pl.BlockSpec((NC, BR, D), lambda i: (0, i, 0)),
            _deg_spec(),
            _full((D, D)), _full((1, D)), _full((1, D)), _full((1, 1)),
        ],
        out_specs=_full((1, 1)),
        out_shape=jax.ShapeDtypeStruct((1, 1), jnp.float32),
        scratch_shapes=[pltpu.VMEM((1, D), jnp.float32)],
    )(agg, ideg, W2, b2, W_ro, b_ro)


def kernel(x, edge_index, W1, b1, W2, b2, W_ro, b_ro):
    src = edge_index[0]
    dst3d = edge_index[1].reshape(NW, NCHUNK, CK)
    odeg, ideg = _sc_degrees(src, edge_index[1])
    x_pad = jnp.pad(x, ((0, NP - N), (0, 0)))
    xp = _tc_scale(x_pad, odeg)
    agg1 = _sc_spmm(xp, src, dst3d)
    h1p = _tc_layer1(agg1, odeg, ideg, W1, b1.reshape(1, D))
    agg2 = _sc_spmm(h1p, src, dst3d)
    out = _tc_readout(agg2, ideg, W2, b2.reshape(1, D),
                      W_ro.reshape(1, D), b_ro.reshape(1, 1))
    return jnp.squeeze(out)


# speedup vs baseline: 10.6778x; 1.0001x over previous
import dataclasses
import functools

import jax
import jax.numpy as jnp
from jax import lax
from jax.experimental import pallas as pl
from jax.experimental.pallas import tpu as pltpu
from jax.experimental.pallas import tpu_sc as plsc

N = 10000
NP = 10240
E = 320000
D = 128

NC = 2
NS = 16
LANES = 16
E_PER_CORE = E // NC
E_PER_TILE = E_PER_CORE // NS
CK = 80
NCHUNK = E_PER_TILE // CK
ROWS_PER_TILE = NP // NS
ZROWS = 128

_MESH = plsc.VectorSubcoreMesh(core_axis_name="c", subcore_axis_name="s")

_SC_CP = pltpu.CompilerParams()
if "needs_layout_passes" in pltpu.CompilerParams.__dataclass_fields__:
    _SC_CP = dataclasses.replace(_SC_CP, needs_layout_passes=False)


NW = NC * NS


def _sc_degrees(src, dst):
    @functools.partial(
        pl.kernel,
        out_type=(
            jax.ShapeDtypeStruct((NW, NP), jnp.float32),
            jax.ShapeDtypeStruct((NW, NP), jnp.float32),
        ),
        mesh=_MESH,
        scratch_types=[
            pltpu.VMEM((E_PER_TILE,), jnp.int32),
            pltpu.VMEM((E_PER_TILE,), jnp.int32),
            pltpu.VMEM((NP,), jnp.float32),
            pltpu.VMEM((NP,), jnp.float32),
            pltpu.SemaphoreType.DMA,
            pltpu.SemaphoreType.DMA,
        ],
        compiler_params=_SC_CP,
    )
    def k(src_hbm, dst_hbm, odeg_hbm, ideg_hbm, sidx, didx, hist_o, hist_i,
          sem_a, sem_b):
        core = lax.axis_index("c")
        sub = lax.axis_index("s")
        wid = core * NS + sub
        base = wid * E_PER_TILE

        pltpu.async_copy(src_hbm.at[pl.ds(base, E_PER_TILE)], sidx, sem_a)
        pltpu.async_copy(dst_hbm.at[pl.ds(base, E_PER_TILE)], didx, sem_b)

        @pl.loop(0, NP // LANES)
        def _(r):
            z = jnp.zeros((LANES,), jnp.float32)
            hist_o[pl.ds(r * LANES, LANES)] = z
            hist_i[pl.ds(r * LANES, LANES)] = z

        pltpu.make_async_copy(src_hbm.at[pl.ds(base, E_PER_TILE)], sidx,
                              sem_a).wait()
        pltpu.make_async_copy(dst_hbm.at[pl.ds(base, E_PER_TILE)], didx,
                              sem_b).wait()
        ones16 = jnp.ones((LANES,), jnp.float32)

        @pl.loop(0, E_PER_TILE // LANES)
        def _(g):
            sl = pl.ds(g * LANES, LANES)
            plsc.addupdate_scatter(hist_o, [sidx[sl]], ones16)
            plsc.addupdate_scatter(hist_i, [didx[sl]], ones16)

        pltpu.sync_copy(hist_o, odeg_hbm.at[wid])
        pltpu.sync_copy(hist_i, ideg_hbm.at[wid])

    return k(src, dst)


RCHUNK = ROWS_PER_TILE // CK


def _sc_spmm(xp, src, dst3d):
    @functools.partial(
        pl.kernel,
        out_type=jax.ShapeDtypeStruct((NC, NP, D), jnp.float32),
        mesh=_MESH,
        scratch_types=[
            pltpu.VMEM((E_PER_TILE,), jnp.int32),
            pltpu.VMEM((NCHUNK, CK), jnp.int32),
            pltpu.VMEM((CK, D), jnp.float32),
            pltpu.VMEM((CK, D), jnp.float32),
            pltpu.VMEM_SHARED((NP, D), jnp.float32),
            pltpu.SemaphoreType.DMA,
            pltpu.SemaphoreType.DMA,
            pltpu.SemaphoreType.DMA,
            pltpu.SemaphoreType.DMA,
        ],
    )
    def k(xp_hbm, src_hbm, dst_hbm, out_hbm, sidx, didx, rows_a, rows_b,
          acc, sem_a, sem_b, sem_sa, sem_sb):
        core = lax.axis_index("c")
        sub = lax.axis_index("s")
        wid = core * NS + sub
        row0 = sub * ROWS_PER_TILE
        base = wid * E_PER_TILE

        pltpu.async_copy(src_hbm.at[pl.ds(base, E_PER_TILE)], sidx, sem_a)
        pltpu.async_copy(dst_hbm.at[wid], didx, sem_b)

        @pl.loop(0, CK)
        def _(r):
            for cc in range(D // LANES):
                rows_a[r, pl.ds(cc * LANES, LANES)] = jnp.zeros(
                    (LANES,), jnp.float32)

        for kk in range(RCHUNK):
            pltpu.sync_copy(rows_a, acc.at[pl.ds(row0 + kk * CK, CK)])
        pltpu.make_async_copy(src_hbm.at[pl.ds(base, E_PER_TILE)], sidx,
                              sem_a).wait()
        pltpu.make_async_copy(dst_hbm.at[wid], didx, sem_b).wait()
        plsc.subcore_barrier()

        class gdesc:
            def __init__(self, ci, buf, sem):
                h = CK // 2
                self.parts = [
                    pltpu.make_async_copy(
                        xp_hbm.at[sidx.at[pl.ds(ci * CK, h)]],
                        buf.at[pl.ds(0, h)], sem),
                    pltpu.make_async_copy(
                        xp_hbm.at[sidx.at[pl.ds(ci * CK + h, h)]],
                        buf.at[pl.ds(h, h)], sem),
                ]

            def start(self):
                for p in self.parts:
                    p.start()

            def wait(self):
                for p in self.parts:
                    p.wait()

        class sdesc:
            def __init__(self, ci, buf, sem):
                self.args = (buf, acc.at[didx.at[ci]], sem)

            def start(self):
                pltpu.async_copy(*self.args, add=True)

            def wait(self):
                pltpu.make_async_copy(*self.args).wait()

        gdesc(0, rows_a, sem_a).start()
        gdesc(1, rows_b, sem_b).start()

        @pl.loop(0, (NCHUNK - 3) // 2)
        def _(i):
            c0 = 2 * i
            gdesc(c0, rows_a, sem_a).wait()
            sdesc(c0, rows_a, sem_sa).start()
            gdesc(c0 + 1, rows_b, sem_b).wait()
            sdesc(c0 + 1, rows_b, sem_sb).start()
            sdesc(c0, rows_a, sem_sa).wait()
            gdesc(c0 + 2, rows_a, sem_a).start()
            sdesc(c0 + 1, rows_b, sem_sb).wait()
            gdesc(c0 + 3, rows_b, sem_b).start()

        gdesc(NCHUNK - 3, rows_a, sem_a).wait()
        sdesc(NCHUNK - 3, rows_a, sem_sa).start()
        gdesc(NCHUNK - 2, rows_b, sem_b).wait()
        sdesc(NCHUNK - 2, rows_b, sem_sb).start()
        sdesc(NCHUNK - 3, rows_a, sem_sa).wait()
        gdesc(NCHUNK - 1, rows_a, sem_a).start()
        gdesc(NCHUNK - 1, rows_a, sem_a).wait()
        sdesc(NCHUNK - 1, rows_a, sem_sa).start()
        sdesc(NCHUNK - 1, rows_a, sem_sa).wait()
        sdesc(NCHUNK - 2, rows_b, sem_sb).wait()

        plsc.subcore_barrier()

        for kk in range(RCHUNK):
            r0 = row0 + kk * CK
            pltpu.sync_copy(acc.at[pl.ds(r0, CK)], rows_a)
            pltpu.sync_copy(rows_a, out_hbm.at[core, pl.ds(r0, CK)])

    return k(xp, src, dst3d)


BR = 1024


def _deg_vec(d_ref):
    return jnp.sum(d_ref[...], axis=0)


def _tc_scale_body(x_ref, od_ref, o_ref):
    ns = lax.rsqrt(jnp.maximum(_deg_vec(od_ref), 1.0))
    o_ref[...] = x_ref[...] * ns[:, None]


def _tc_layer1_body(agg_ref, od_ref, id_ref, w_ref, b_ref, o_ref):
    agg = agg_ref[0] + agg_ref[1]
    nd = lax.rsqrt(jnp.maximum(_deg_vec(id_ref), 1.0))
    ns = lax.rsqrt(jnp.maximum(_deg_vec(od_ref), 1.0))
    h = jnp.dot(agg * nd[:, None], w_ref[...],
                preferred_element_type=jnp.float32) + b_ref[...]
    o_ref[...] = jnp.maximum(h, 0.0) * ns[:, None]


def _tc_readout_body(agg_ref, id_ref, w2_ref, b2_ref, wro_ref, bro_ref,
                     o_ref, acc_ref):
    i = pl.program_id(0)
    agg = agg_ref[0] + agg_ref[1]
    nd = lax.rsqrt(jnp.maximum(_deg_vec(id_ref), 1.0))
    part = jnp.sum(agg * nd[:, None], axis=0, keepdims=True)

    @pl.when(i == 0)
    def _():
        acc_ref[...] = part

    @pl.when(i > 0)
    def _():
        acc_ref[...] = acc_ref[...] + part

    @pl.when(i == pl.num_programs(0) - 1)
    def _():
        g = jnp.dot(acc_ref[...] * (1.0 / N), w2_ref[...],
                    preferred_element_type=jnp.float32) + b2_ref[...]
        z = jnp.sum(g * wro_ref[...], axis=1, keepdims=True) + bro_ref[...]
        o_ref[...] = jax.nn.sigmoid(z)


def _deg_spec():
    return pl.BlockSpec((NW, BR), lambda i: (0, i))


def _full(shape):
    return pl.BlockSpec(shape, lambda i: tuple(0 for _ in shape))


def _tc_scale(x, odeg):
    return pl.pallas_call(
        _tc_scale_body,
        grid=(NP // BR,),
        in_specs=[pl.BlockSpec((BR, D), lambda i: (i, 0)), _deg_spec()],
        out_specs=pl.BlockSpec((BR, D), lambda i: (i, 0)),
        out_shape=jax.ShapeDtypeStruct((NP, D), jnp.float32),
    )(x, odeg)


def _tc_layer1(agg, odeg, ideg, W1, b1):
    return pl.pallas_call(
        _tc_layer1_body,
        grid=(NP // BR,),
        in_specs=[
            pl.BlockSpec((NC, BR, D), lambda i: (0, i, 0)),
            _deg_spec(), _deg_spec(),
            _full((D, D)), _full((1, D)),
        ],
        out_specs=pl.BlockSpec((BR, D), lambda i: (i, 0)),
        out_shape=jax.ShapeDtypeStruct((NP, D), jnp.float32),
    )(agg, odeg, ideg, W1, b1)


def _tc_readout(agg, ideg, W2, b2, W_ro, b_ro):
    return pl.pallas_call(
        _tc_readout_body,
        grid=(NP // BR,),
        in_specs=[
            pl.BlockSpec((NC, BR, D), lambda i: (0, i, 0)),
            _deg_spec(),
            _full((D, D)), _full((1, D)), _full((1, D)), _full((1, 1)),
        ],
        out_specs=_full((1, 1)),
        out_shape=jax.ShapeDtypeStruct((1, 1), jnp.float32),
        scratch_shapes=[pltpu.VMEM((1, D), jnp.float32)],
    )(agg, ideg, W2, b2, W_ro, b_ro)


def kernel(x, edge_index, W1, b1, W2, b2, W_ro, b_ro):
    src = edge_index[0]
    dst3d = edge_index[1].reshape(NW, NCHUNK, CK)
    odeg, ideg = _sc_degrees(src, edge_index[1])
    x_pad = jnp.pad(x, ((0, NP - N), (0, 0)))
    xp = _tc_scale(x_pad, odeg)
    agg1 = _sc_spmm(xp, src, dst3d)
    h1p = _tc_layer1(agg1, odeg, ideg, W1, b1.reshape(1, D))
    agg2 = _sc_spmm(h1p, src, dst3d)
    out = _tc_readout(agg2, ideg, W2, b2.reshape(1, D),
                      W_ro.reshape(1, D), b_ro.reshape(1, 1))
    return jnp.squeeze(out)


# speedup vs baseline: 10.6944x; 1.0016x over previous
import dataclasses
import functools

import jax
import jax.numpy as jnp
from jax import lax
from jax.experimental import pallas as pl
from jax.experimental.pallas import tpu as pltpu
from jax.experimental.pallas import tpu_sc as plsc

N = 10000
NP = 10240
E = 320000
D = 128

NC = 2
NS = 16
LANES = 16
E_PER_CORE = E // NC
E_PER_TILE = E_PER_CORE // NS
CK = 80
NCHUNK = E_PER_TILE // CK
ROWS_PER_TILE = NP // NS

_MESH = plsc.VectorSubcoreMesh(core_axis_name="c", subcore_axis_name="s")

_SC_CP = pltpu.CompilerParams()
if "needs_layout_passes" in pltpu.CompilerParams.__dataclass_fields__:
    _SC_CP = dataclasses.replace(_SC_CP, needs_layout_passes=False)


NW = NC * NS


def _sc_degrees(src, dst):
    @functools.partial(
        pl.kernel,
        out_type=(
            jax.ShapeDtypeStruct((NW, NP), jnp.float32),
            jax.ShapeDtypeStruct((NW, NP), jnp.float32),
        ),
        mesh=_MESH,
        scratch_types=[
            pltpu.VMEM((E_PER_TILE,), jnp.int32),
            pltpu.VMEM((E_PER_TILE,), jnp.int32),
            pltpu.VMEM((NP,), jnp.float32),
            pltpu.VMEM((NP,), jnp.float32),
            pltpu.SemaphoreType.DMA,
            pltpu.SemaphoreType.DMA,
        ],
        compiler_params=_SC_CP,
    )
    def k(src_hbm, dst_hbm, odeg_hbm, ideg_hbm, sidx, didx, hist_o, hist_i,
          sem_a, sem_b):
        core = lax.axis_index("c")
        sub = lax.axis_index("s")
        wid = core * NS + sub
        base = wid * E_PER_TILE

        pltpu.async_copy(src_hbm.at[pl.ds(base, E_PER_TILE)], sidx, sem_a)
        pltpu.async_copy(dst_hbm.at[pl.ds(base, E_PER_TILE)], didx, sem_b)

        @pl.loop(0, NP // LANES)
        def _(r):
            z = jnp.zeros((LANES,), jnp.float32)
            hist_o[pl.ds(r * LANES, LANES)] = z
            hist_i[pl.ds(r * LANES, LANES)] = z

        pltpu.make_async_copy(src_hbm.at[pl.ds(base, E_PER_TILE)], sidx,
                              sem_a).wait()
        pltpu.make_async_copy(dst_hbm.at[pl.ds(base, E_PER_TILE)], didx,
                              sem_b).wait()
        ones16 = jnp.ones((LANES,), jnp.float32)

        @pl.loop(0, E_PER_TILE // LANES)
        def _(g):
            sl = pl.ds(g * LANES, LANES)
            plsc.addupdate_scatter(hist_o, [sidx[sl]], ones16)
            plsc.addupdate_scatter(hist_i, [didx[sl]], ones16)

        pltpu.sync_copy(hist_o, odeg_hbm.at[wid])
        pltpu.sync_copy(hist_i, ideg_hbm.at[wid])

    return k(src, dst)


RCHUNK = ROWS_PER_TILE // CK


def _sc_spmm(xp, src, dst3d):
    @functools.partial(
        pl.kernel,
        out_type=jax.ShapeDtypeStruct((NC, NP, D), jnp.float32),
        mesh=_MESH,
        scratch_types=[
            pltpu.VMEM((E_PER_TILE,), jnp.int32),
            pltpu.VMEM((NCHUNK, CK), jnp.int32),
            pltpu.VMEM((CK, D), jnp.float32),
            pltpu.VMEM((CK, D), jnp.float32),
            pltpu.VMEM_SHARED((NP, D), jnp.float32),
            pltpu.SemaphoreType.DMA,
            pltpu.SemaphoreType.DMA,
            pltpu.SemaphoreType.DMA,
            pltpu.SemaphoreType.DMA,
        ],
    )
    def k(xp_hbm, src_hbm, dst_hbm, out_hbm, sidx, didx, rows_a, rows_b,
          acc, sem_a, sem_b, sem_sa, sem_sb):
        core = lax.axis_index("c")
        sub = lax.axis_index("s")
        wid = core * NS + sub
        row0 = sub * ROWS_PER_TILE
        base = wid * E_PER_TILE

        pltpu.async_copy(src_hbm.at[pl.ds(base, E_PER_TILE)], sidx, sem_a)
        pltpu.async_copy(dst_hbm.at[wid], didx, sem_b)

        @pl.loop(0, CK)
        def _(r):
            for cc in range(D // LANES):
                rows_a[r, pl.ds(cc * LANES, LANES)] = jnp.zeros(
                    (LANES,), jnp.float32)

        for kk in range(RCHUNK):
            pltpu.sync_copy(rows_a, acc.at[pl.ds(row0 + kk * CK, CK)])
        pltpu.make_async_copy(src_hbm.at[pl.ds(base, E_PER_TILE)], sidx,
                              sem_a).wait()
        pltpu.make_async_copy(dst_hbm.at[wid], didx, sem_b).wait()
        plsc.subcore_barrier()

        class gdesc:
            def __init__(self, ci, buf, sem):
                h = CK // 2
                self.parts = [
                    pltpu.make_async_copy(
                        xp_hbm.at[sidx.at[pl.ds(ci * CK, h)]],
                        buf.at[pl.ds(0, h)], sem),
                    pltpu.make_async_copy(
                        xp_hbm.at[sidx.at[pl.ds(ci * CK + h, h)]],
                        buf.at[pl.ds(h, h)], sem),
                ]

            def start(self):
                for p in self.parts:
                    p.start()

            def wait(self):
                for p in self.parts:
                    p.wait()

        class sdesc:
            def __init__(self, ci, buf, sem):
                self.args = (buf, acc.at[didx.at[ci]], sem)

            def start(self):
                pltpu.async_copy(*self.args, add=True)

            def wait(self):
                pltpu.make_async_copy(*self.args).wait()

        gdesc(0, rows_a, sem_a).start()
        gdesc(1, rows_b, sem_b).start()

        @pl.loop(0, (NCHUNK - 3) // 2)
        def _(i):
            c0 = 2 * i
            gdesc(c0, rows_a, sem_a).wait()
            sdesc(c0, rows_a, sem_sa).start()
            gdesc(c0 + 1, rows_b, sem_b).wait()
            sdesc(c0 + 1, rows_b, sem_sb).start()
            sdesc(c0, rows_a, sem_sa).wait()
            gdesc(c0 + 2, rows_a, sem_a).start()
            sdesc(c0 + 1, rows_b, sem_sb).wait()
            gdesc(c0 + 3, rows_b, sem_b).start()

        gdesc(NCHUNK - 3, rows_a, sem_a).wait()
        sdesc(NCHUNK - 3, rows_a, sem_sa).start()
        gdesc(NCHUNK - 2, rows_b, sem_b).wait()
        sdesc(NCHUNK - 2, rows_b, sem_sb).start()
        sdesc(NCHUNK - 3, rows_a, sem_sa).wait()
        gdesc(NCHUNK - 1, rows_a, sem_a).start()
        gdesc(NCHUNK - 1, rows_a, sem_a).wait()
        sdesc(NCHUNK - 1, rows_a, sem_sa).start()
        sdesc(NCHUNK - 1, rows_a, sem_sa).wait()
        sdesc(NCHUNK - 2, rows_b, sem_sb).wait()

        plsc.subcore_barrier()

        for kk in range(RCHUNK):
            r0 = row0 + kk * CK
            pltpu.sync_copy(acc.at[pl.ds(r0, CK)], rows_a)
            pltpu.sync_copy(rows_a, out_hbm.at[core, pl.ds(r0, CK)])

    return k(xp, src, dst3d)


BR = 1024


def _deg_vec(d_ref):
    return jnp.sum(d_ref[...], axis=0)


def _tc_scale_body(x_ref, od_ref, o_ref):
    ns = lax.rsqrt(jnp.maximum(_deg_vec(od_ref), 1.0))
    o_ref[...] = x_ref[...] * ns[:, None]


def _tc_layer1_body(agg_ref, od_ref, id_ref, w_ref, b_ref, o_ref):
    agg = agg_ref[0] + agg_ref[1]
    nd = lax.rsqrt(jnp.maximum(_deg_vec(id_ref), 1.0))
    ns = lax.rsqrt(jnp.maximum(_deg_vec(od_ref), 1.0))
    h = jnp.dot(agg * nd[:, None], w_ref[...],
                preferred_element_type=jnp.float32) + b_ref[...]
    o_ref[...] = jnp.maximum(h, 0.0) * ns[:, None]


def _tc_readout_body(agg_ref, id_ref, w2_ref, b2_ref, wro_ref, bro_ref,
                     o_ref, acc_ref):
    i = pl.program_id(0)
    agg = agg_ref[0] + agg_ref[1]
    nd = lax.rsqrt(jnp.maximum(_deg_vec(id_ref), 1.0))
    part = jnp.sum(agg * nd[:, None], axis=0, keepdims=True)

    @pl.when(i == 0)
    def _():
        acc_ref[...] = part

    @pl.when(i > 0)
    def _():
        acc_ref[...] = acc_ref[...] + part

    @pl.when(i == pl.num_programs(0) - 1)
    def _():
        g = jnp.dot(acc_ref[...] * (1.0 / N), w2_ref[...],
                    preferred_element_type=jnp.float32) + b2_ref[...]
        z = jnp.sum(g * wro_ref[...], axis=1, keepdims=True) + bro_ref[...]
        o_ref[...] = jax.nn.sigmoid(z)


def _deg_spec():
    return pl.BlockSpec((NW, BR), lambda i: (0, i))


def _full(shape):
    return pl.BlockSpec(shape, lambda i: tuple(0 for _ in shape))


def _tc_scale(x, odeg):
    return pl.pallas_call(
        _tc_scale_body,
        grid=(NP // BR,),
        in_specs=[pl.BlockSpec((BR, D), lambda i: (i, 0)), _deg_spec()],
        out_specs=pl.BlockSpec((BR, D), lambda i: (i, 0)),
        out_shape=jax.ShapeDtypeStruct((NP, D), jnp.float32),
    )(x, odeg)


def _tc_layer1(agg, odeg, ideg, W1, b1):
    return pl.pallas_call(
        _tc_layer1_body,
        grid=(NP // BR,),
        in_specs=[
            pl.BlockSpec((NC, BR, D), lambda i: (0, i, 0)),
            _deg_spec(), _deg_spec(),
            _full((D, D)), _full((1, D)),
        ],
        out_specs=pl.BlockSpec((BR, D), lambda i: (i, 0)),
        out_shape=jax.ShapeDtypeStruct((NP, D), jnp.float32),
    )(agg, odeg, ideg, W1, b1)


def _tc_readout(agg, ideg, W2, b2, W_ro, b_ro):
    return pl.pallas_call(
        _tc_readout_body,
        grid=(NP // BR,),
        in_specs=[
            pl.BlockSpec((NC, BR, D), lambda i: (0, i, 0)),
            _deg_spec(),
            _full((D, D)), _full((1, D)), _full((1, D)), _full((1, 1)),
        ],
        out_specs=_full((1, 1)),
        out_shape=jax.ShapeDtypeStruct((1, 1), jnp.float32),
        scratch_shapes=[pltpu.VMEM((1, D), jnp.float32)],
    )(agg, ideg, W2, b2, W_ro, b_ro)


def kernel(x, edge_index, W1, b1, W2, b2, W_ro, b_ro):
    src = edge_index[0]
    dst3d = edge_index[1].reshape(NW, NCHUNK, CK)
    odeg, ideg = _sc_degrees(src, edge_index[1])
    x_pad = jnp.pad(x, ((0, NP - N), (0, 0)))
    xp = _tc_scale(x_pad, odeg)
    agg1 = _sc_spmm(xp, src, dst3d)
    h1p = _tc_layer1(agg1, odeg, ideg, W1, b1.reshape(1, D))
    agg2 = _sc_spmm(h1p, src, dst3d)
    out = _tc_readout(agg2, ideg, W2, b2.reshape(1, D),
                      W_ro.reshape(1, D), b_ro.reshape(1, 1))
    return jnp.squeeze(out)
